# baseline (speedup 1.0000x reference)
import functools

import jax
import jax.numpy as jnp
from jax import lax
from jax.experimental import pallas as pl
from jax.experimental.pallas import tpu as pltpu
from jax.experimental.pallas import tpu_sc as plsc

B = 16384
D = 129
W = 128
NB = 1000000
TAIL = (NB // W) * W
TW = NB - TAIL
NC, NS = 2, 16
NW = NC * NS
BPW = B // NW
R = 2
STEPS = BPW // R

_mesh = plsc.VectorSubcoreMesh(core_axis_name="c", subcore_axis_name="s")


@functools.partial(
    pl.kernel,
    mesh=_mesh,
    out_type=jax.ShapeDtypeStruct((B,), jnp.float32),
    compiler_params=pltpu.CompilerParams(needs_layout_passes=False),
    scratch_types=[
        pltpu.VMEM((BPW,), jnp.int32),
        pltpu.VMEM((BPW,), jnp.int32),
        pltpu.VMEM((D, W), jnp.float32),
        pltpu.VMEM((D, W), jnp.float32),
        pltpu.VMEM((D, W), jnp.float32),
        pltpu.VMEM((D, W), jnp.float32),
        pltpu.VMEM((D, TW), jnp.float32),
        pltpu.VMEM((BPW,), jnp.float32),
        pltpu.SemaphoreType.DMA,
        pltpu.SemaphoreType.DMA,
        pltpu.SemaphoreType.DMA,
        pltpu.SemaphoreType.DMA,
    ],
)
def _sc_dot2(users_hbm, items_hbm, uT_hbm, iT_hbm, out_hbm,
             uv, iv, ublk0, ublk1, iblk0, iblk1, tailb, outc,
             sem_u0, sem_u1, sem_i0, sem_i1):
    wid = lax.axis_index("s") * NC + lax.axis_index("c")
    base = wid * BPW
    lane = jnp.arange(16, dtype=jnp.int32)
    d128 = jnp.full((16,), 128, jnp.int32)

    pltpu.sync_copy(users_hbm.at[pl.ds(base, BPW)], uv)
    pltpu.sync_copy(items_hbm.at[pl.ds(base, BPW)], iv)

    sem_u = (sem_u0, sem_u1)
    sem_i = (sem_i0, sem_i1)
    ublk = (ublk0, ublk1)
    iblk = (iblk0, iblk1)

    def sread(ref, b):
        return plsc.load_gather(ref, [jnp.zeros((16,), jnp.int32) + b])[0]

    def pair_normal(b):
        ru = sread(uv, b)
        ri = sread(iv, b)
        return (ru < TAIL) & (ri < TAIL), ru, ri

    def issue(b, k):
        ok, ru, ri = pair_normal(b)

        @pl.when(ok)
        def _():
            cu = pl.multiple_of((ru // W) * W, W)
            ci = pl.multiple_of((ri // W) * W, W)
            pltpu.async_copy(uT_hbm.at[pl.ds(0, 64), pl.ds(cu, W)],
                             ublk[k].at[pl.ds(0, 64)], sem_u[k])
            pltpu.async_copy(uT_hbm.at[pl.ds(64, D - 64), pl.ds(cu, W)],
                             ublk[k].at[pl.ds(64, D - 64)], sem_u[k])
            pltpu.async_copy(iT_hbm.at[pl.ds(0, 64), pl.ds(ci, W)],
                             iblk[k].at[pl.ds(0, 64)], sem_i[k])
            pltpu.async_copy(iT_hbm.at[pl.ds(64, D - 64), pl.ds(ci, W)],
                             iblk[k].at[pl.ds(64, D - 64)], sem_i[k])

    def extract(r, blk):
        col = jnp.zeros((16,), jnp.int32) + (r % W)
        pieces = []
        for j in range(8):
            pieces.append(plsc.load_gather(blk, [lane + j * 16, col]))
        pieces.append(plsc.load_gather(blk, [d128, col]))
        return pieces

    def dot(up, ip):
        acc = up[0] * ip[0]
        for j in range(1, 8):
            acc = acc + up[j] * ip[j]
        acc = acc + jnp.where(lane == 0, up[8] * ip[8], 0.0)
        return jnp.sum(acc)

    for k in range(R):
        issue(k, k)

    def step(t, res):
        for k in range(R):
            b = t * R + k
            ok, ru, ri = pair_normal(b)

            @pl.when(ok)
            def _():
                pltpu.make_async_copy(
                    uT_hbm.at[:, pl.ds(0, W)], ublk[k], sem_u[k]).wait()
                pltpu.make_async_copy(
                    iT_hbm.at[:, pl.ds(0, W)], iblk[k], sem_i[k]).wait()

            up = extract(ru, ublk[k])
            ip = extract(ri, iblk[k])
            s = dot(up, ip)

            @pl.when(b + R < BPW)
            def _():
                issue(b + R, k)

            res = jnp.where(ok & (lane == (b % 16)), s, res)

        @pl.when(t % 8 == 7)
        def _():
            outc[pl.ds((t // 8) * 16, 16)] = res

        return res

    lax.fori_loop(0, STEPS, step, jnp.zeros((16,), jnp.float32))

    def fetch_one(r, tbl):
        @pl.when(r < TAIL)
        def _():
            cb = pl.multiple_of((r // W) * W, W)
            pltpu.sync_copy(tbl.at[:, pl.ds(cb, W)], ublk0)

        @pl.when(r >= TAIL)
        def _():
            pltpu.sync_copy(tbl.at[:, pl.ds(TAIL, TW)], tailb)

        colm = jnp.zeros((16,), jnp.int32) + (r % W)
        colt = jnp.clip(jnp.zeros((16,), jnp.int32) + (r - TAIL), 0, TW - 1)
        pieces = []
        for j in range(8):
            m = plsc.load_gather(ublk0, [lane + j * 16, colm])
            tl = plsc.load_gather(tailb, [lane + j * 16, colt])
            pieces.append(jnp.where(r < TAIL, m, tl))
        m = plsc.load_gather(ublk0, [d128, colm])
        tl = plsc.load_gather(tailb, [d128, colt])
        pieces.append(jnp.where(r < TAIL, m, tl))
        return pieces

    def ep_step(b, carry):
        ok, ru, ri = pair_normal(b)

        @pl.when(jnp.logical_not(ok))
        def _():
            up = fetch_one(ru, uT_hbm)
            ip = fetch_one(ri, iT_hbm)
            s = dot(up, ip)
            gb = (b // 16) * 16
            old = outc[pl.ds(gb, 16)]
            outc[pl.ds(gb, 16)] = jnp.where(lane == (b % 16), s, old)

        return carry

    lax.fori_loop(0, BPW, ep_step, 0)
    pltpu.sync_copy(outc, out_hbm.at[pl.ds(base, BPW)])


def kernel(users, items, user_emb, item_emb):
    perm = jnp.argsort(users)
    tmp = _sc_dot2(users[perm], items[perm], user_emb.T, item_emb.T)
    return jnp.zeros((B,), jnp.float32).at[perm].set(tmp)


# speedup vs baseline: 1.2322x; 1.2322x over previous
import functools

import jax
import jax.numpy as jnp
from jax import lax
from jax.experimental import pallas as pl
from jax.experimental.pallas import tpu as pltpu
from jax.experimental.pallas import tpu_sc as plsc

B = 16384
D = 129
W = 128
NB = 1000000
TAIL = (NB // W) * W
TW = NB - TAIL
NC, NS = 2, 16
NW = NC * NS
BPW = B // NW
R = 2
STEPS = BPW // R

_mesh = plsc.VectorSubcoreMesh(core_axis_name="c", subcore_axis_name="s")


@functools.partial(
    pl.kernel,
    mesh=_mesh,
    out_type=jax.ShapeDtypeStruct((B,), jnp.float32),
    compiler_params=pltpu.CompilerParams(needs_layout_passes=False),
    scratch_types=[
        pltpu.VMEM((BPW,), jnp.int32),
        pltpu.VMEM((BPW,), jnp.int32),
        pltpu.VMEM((D, W), jnp.float32),
        pltpu.VMEM((D, W), jnp.float32),
        pltpu.VMEM((D, W), jnp.float32),
        pltpu.VMEM((D, W), jnp.float32),
        pltpu.VMEM((D, TW), jnp.float32),
        pltpu.VMEM((BPW,), jnp.int32),
        pltpu.VMEM((BPW,), jnp.int32),
        pltpu.VMEM((BPW,), jnp.float32),
        pltpu.SemaphoreType.DMA,
        pltpu.SemaphoreType.DMA,
        pltpu.SemaphoreType.DMA,
        pltpu.SemaphoreType.DMA,
    ],
)
def _sc_dot2(users_hbm, items_hbm, uT_hbm, iT_hbm, out_hbm,
             uv, iv, ublk0, ublk1, iblk0, iblk1, tailb, flagv, nwv, outc,
             sem_u0, sem_u1, sem_i0, sem_i1):
    wid = lax.axis_index("s") * NC + lax.axis_index("c")
    base = wid * BPW
    lane = jnp.arange(16, dtype=jnp.int32)
    d128 = jnp.full((16,), 128, jnp.int32)

    pltpu.sync_copy(users_hbm.at[pl.ds(base, BPW)], uv)
    pltpu.sync_copy(items_hbm.at[pl.ds(base, BPW)], iv)

    sem_u = (sem_u0, sem_u1)
    sem_i = (sem_i0, sem_i1)
    ublk = (ublk0, ublk1)
    iblk = (iblk0, iblk1)

    carryw = jnp.int32(-1)
    carrynw = jnp.int32(-1)
    for c in range(BPW // 16):
        wv = jnp.minimum(uv[pl.ds(c * 16, 16)] // W, TAIL // W - 1)
        prev = plsc.load_gather(
            uv, [jnp.maximum(jnp.zeros((16,), jnp.int32) + c * 16 + lane - 1, 0)])
        wsh = jnp.where(lane == 0, carryw,
                        jnp.minimum(prev // W, TAIL // W - 1))
        fl = (wv != wsh).astype(jnp.int32)
        nw = jnp.cumsum(fl) + carrynw
        flagv[pl.ds(c * 16, 16)] = fl
        nwv[pl.ds(c * 16, 16)] = nw
        carryw = wv[15]
        carrynw = nw[15]

    def sread(ref, b):
        return plsc.load_gather(ref, [jnp.zeros((16,), jnp.int32) + b])[0]

    def pair_normal(b):
        ru = sread(uv, b)
        ri = sread(iv, b)
        return (ru < TAIL) & (ri < TAIL), ru, ri

    def issue_u(b):
        fl = sread(flagv, b)
        par = sread(nwv, b) % 2
        ru = sread(uv, b)
        cu = pl.multiple_of((jnp.minimum(ru // W, TAIL // W - 1)) * W, W)

        @pl.when((fl == 1) & (par == 0))
        def _():
            pltpu.async_copy(uT_hbm.at[:, pl.ds(cu, W)], ublk0, sem_u0)

        @pl.when((fl == 1) & (par == 1))
        def _():
            pltpu.async_copy(uT_hbm.at[:, pl.ds(cu, W)], ublk1, sem_u1)

    def issue_i(b, k):
        ri = sread(iv, b)

        @pl.when(ri < TAIL)
        def _():
            ci = pl.multiple_of((ri // W) * W, W)
            pltpu.async_copy(iT_hbm.at[:, pl.ds(ci, W)], iblk[k], sem_i[k])

    def extract(r, blk):
        col = jnp.zeros((16,), jnp.int32) + (r % W)
        pieces = []
        for j in range(8):
            pieces.append(plsc.load_gather(blk, [lane + j * 16, col]))
        pieces.append(plsc.load_gather(blk, [d128, col]))
        return pieces

    def extract_u(b, r):
        par = sread(nwv, b) % 2
        p0 = extract(r, ublk0)
        p1 = extract(r, ublk1)
        return [jnp.where(par == 0, a, bb) for a, bb in zip(p0, p1)]

    def dot(up, ip):
        acc = up[0] * ip[0]
        for j in range(1, 8):
            acc = acc + up[j] * ip[j]
        acc = acc + jnp.where(lane == 0, up[8] * ip[8], 0.0)
        return jnp.sum(acc)

    for k in range(R):
        issue_u(k)
        issue_i(k, k)

    def step(t, res):
        for k in range(R):
            b = t * R + k
            ok, ru, ri = pair_normal(b)
            fl = sread(flagv, b)
            par = sread(nwv, b) % 2

            @pl.when((fl == 1) & (par == 0))
            def _():
                pltpu.make_async_copy(
                    uT_hbm.at[:, pl.ds(0, W)], ublk0, sem_u0).wait()

            @pl.when((fl == 1) & (par == 1))
            def _():
                pltpu.make_async_copy(
                    uT_hbm.at[:, pl.ds(0, W)], ublk1, sem_u1).wait()

            @pl.when(ri < TAIL)
            def _():
                pltpu.make_async_copy(
                    iT_hbm.at[:, pl.ds(0, W)], iblk[k], sem_i[k]).wait()

            up = extract_u(b, ru)
            ip = extract(ri, iblk[k])
            s = dot(up, ip)

            @pl.when(b + R < BPW)
            def _():
                issue_u(b + R)
                issue_i(b + R, k)

            res = jnp.where(ok & (lane == (b % 16)), s, res)

        @pl.when(t % 8 == 7)
        def _():
            outc[pl.ds((t // 8) * 16, 16)] = res

        return res

    lax.fori_loop(0, STEPS, step, jnp.zeros((16,), jnp.float32))

    def fetch_one(r, tbl):
        @pl.when(r < TAIL)
        def _():
            cb = pl.multiple_of((r // W) * W, W)
            pltpu.sync_copy(tbl.at[:, pl.ds(cb, W)], ublk0)

        @pl.when(r >= TAIL)
        def _():
            pltpu.sync_copy(tbl.at[:, pl.ds(TAIL, TW)], tailb)

        colm = jnp.zeros((16,), jnp.int32) + (r % W)
        colt = jnp.clip(jnp.zeros((16,), jnp.int32) + (r - TAIL), 0, TW - 1)
        pieces = []
        for j in range(8):
            m = plsc.load_gather(ublk0, [lane + j * 16, colm])
            tl = plsc.load_gather(tailb, [lane + j * 16, colt])
            pieces.append(jnp.where(r < TAIL, m, tl))
        m = plsc.load_gather(ublk0, [d128, colm])
        tl = plsc.load_gather(tailb, [d128, colt])
        pieces.append(jnp.where(r < TAIL, m, tl))
        return pieces

    def ep_step(b, carry):
        ok, ru, ri = pair_normal(b)

        @pl.when(jnp.logical_not(ok))
        def _():
            up = fetch_one(ru, uT_hbm)
            ip = fetch_one(ri, iT_hbm)
            s = dot(up, ip)
            gb = (b // 16) * 16
            old = outc[pl.ds(gb, 16)]
            outc[pl.ds(gb, 16)] = jnp.where(lane == (b % 16), s, old)

        return carry

    lax.fori_loop(0, BPW, ep_step, 0)
    pltpu.sync_copy(outc, out_hbm.at[pl.ds(base, BPW)])


def kernel(users, items, user_emb, item_emb):
    perm = jnp.argsort(users)
    tmp = _sc_dot2(users[perm], items[perm], user_emb.T, item_emb.T)
    return jnp.zeros((B,), jnp.float32).at[perm].set(tmp)


# speedup vs baseline: 1.2326x; 1.0003x over previous
import functools

import jax
import jax.numpy as jnp
from jax import lax
from jax.experimental import pallas as pl
from jax.experimental.pallas import tpu as pltpu
from jax.experimental.pallas import tpu_sc as plsc

B = 16384
D = 129
W = 128
NB = 1000000
TAIL = (NB // W) * W
TW = NB - TAIL
NC, NS = 2, 16
NW = NC * NS
BPW = B // NW
R = 2
STEPS = BPW // R

_mesh = plsc.VectorSubcoreMesh(core_axis_name="c", subcore_axis_name="s")


@functools.partial(
    pl.kernel,
    mesh=_mesh,
    out_type=jax.ShapeDtypeStruct((B,), jnp.float32),
    compiler_params=pltpu.CompilerParams(needs_layout_passes=False),
    scratch_types=[
        pltpu.VMEM((BPW,), jnp.int32),
        pltpu.VMEM((BPW,), jnp.int32),
        pltpu.VMEM((D, W), jnp.float32),
        pltpu.VMEM((D, W), jnp.float32),
        pltpu.VMEM((D, W), jnp.float32),
        pltpu.VMEM((D, W), jnp.float32),
        pltpu.VMEM((D, TW), jnp.float32),
        pltpu.VMEM((BPW,), jnp.int32),
        pltpu.VMEM((BPW,), jnp.int32),
        pltpu.VMEM((BPW,), jnp.float32),
        pltpu.SemaphoreType.DMA,
        pltpu.SemaphoreType.DMA,
        pltpu.SemaphoreType.DMA,
        pltpu.SemaphoreType.DMA,
    ],
)
def _sc_dot2(users_hbm, items_hbm, uT_hbm, iT_hbm, out_hbm,
             uv, iv, ublk0, ublk1, iblk0, iblk1, tailb, flagv, nwv, outc,
             sem_u0, sem_u1, sem_i0, sem_i1):
    wid = lax.axis_index("s") * NC + lax.axis_index("c")
    base = wid * BPW
    lane = jnp.arange(16, dtype=jnp.int32)
    d128 = jnp.full((16,), 128, jnp.int32)

    pltpu.sync_copy(users_hbm.at[pl.ds(base, BPW)], uv)
    pltpu.sync_copy(items_hbm.at[pl.ds(base, BPW)], iv)

    sem_i = (sem_i0, sem_i1)
    iblk = (iblk0, iblk1)

    carryw = jnp.int32(-1)
    carrynw = jnp.int32(-1)
    for c in range(BPW // 16):
        wv = jnp.minimum(uv[pl.ds(c * 16, 16)] // W, TAIL // W - 1)
        prev = plsc.load_gather(
            uv, [jnp.maximum(jnp.zeros((16,), jnp.int32) + c * 16 + lane - 1, 0)])
        wsh = jnp.where(lane == 0, carryw,
                        jnp.minimum(prev // W, TAIL // W - 1))
        fl = (wv != wsh).astype(jnp.int32)
        nw = jnp.cumsum(fl) + carrynw
        flagv[pl.ds(c * 16, 16)] = fl
        nwv[pl.ds(c * 16, 16)] = nw
        carryw = wv[15]
        carrynw = nw[15]

    def sread(ref, b):
        return plsc.load_gather(ref, [jnp.zeros((16,), jnp.int32) + b])[0]

    def pair_normal(b):
        ru = sread(uv, b)
        ri = sread(iv, b)
        return (ru < TAIL) & (ri < TAIL), ru, ri

    def issue_u(b):
        fl = sread(flagv, b)
        par = sread(nwv, b) % 2
        ru = sread(uv, b)
        cu = pl.multiple_of((jnp.minimum(ru // W, TAIL // W - 1)) * W, W)

        @pl.when((fl == 1) & (par == 0))
        def _():
            pltpu.async_copy(uT_hbm.at[:, pl.ds(cu, W)], ublk0, sem_u0)

        @pl.when((fl == 1) & (par == 1))
        def _():
            pltpu.async_copy(uT_hbm.at[:, pl.ds(cu, W)], ublk1, sem_u1)

    def issue_i(b, k):
        ri = sread(iv, b)

        @pl.when(ri < TAIL)
        def _():
            ci = pl.multiple_of((ri // W) * W, W)
            pltpu.async_copy(iT_hbm.at[:, pl.ds(ci, W)], iblk[k], sem_i[k])

    def extract(r, blk):
        col = jnp.zeros((16,), jnp.int32) + (r % W)
        pieces = []
        for j in range(8):
            pieces.append(plsc.load_gather(blk, [lane + j * 16, col]))
        pieces.append(plsc.load_gather(blk, [d128, col]))
        return pieces

    def extract_u(b, r):
        par = sread(nwv, b) % 2
        p0 = extract(r, ublk0)
        p1 = extract(r, ublk1)
        return [jnp.where(par == 0, a, bb) for a, bb in zip(p0, p1)]

    def dot(up, ip):
        acc = up[0] * ip[0]
        for j in range(1, 8):
            acc = acc + up[j] * ip[j]
        acc = acc + jnp.where(lane == 0, up[8] * ip[8], 0.0)
        return jnp.sum(acc)

    for k in range(R):
        issue_u(k)
        issue_i(k, k)

    def step(t, res):
        for k in range(R):
            b = t * R + k
            ok, ru, ri = pair_normal(b)
            fl = sread(flagv, b)
            par = sread(nwv, b) % 2

            @pl.when((fl == 1) & (par == 0))
            def _():
                pltpu.make_async_copy(
                    uT_hbm.at[:, pl.ds(0, W)], ublk0, sem_u0).wait()

            @pl.when((fl == 1) & (par == 1))
            def _():
                pltpu.make_async_copy(
                    uT_hbm.at[:, pl.ds(0, W)], ublk1, sem_u1).wait()

            @pl.when(ri < TAIL)
            def _():
                pltpu.make_async_copy(
                    iT_hbm.at[:, pl.ds(0, W)], iblk[k], sem_i[k]).wait()

            up = extract_u(b, ru)
            ip = extract(ri, iblk[k])
            s = dot(up, ip)

            @pl.when(b + R < BPW)
            def _():
                issue_u(b + R)
                issue_i(b + R, k)

            res = jnp.where(ok & (lane == (b % 16)), s, res)

        @pl.when(t % 8 == 7)
        def _():
            outc[pl.ds((t // 8) * 16, 16)] = res

        return res

    lax.fori_loop(0, STEPS, step, jnp.zeros((16,), jnp.float32))

    def fetch_one(r, tbl):
        @pl.when(r < TAIL)
        def _():
            cb = pl.multiple_of((r // W) * W, W)
            pltpu.sync_copy(tbl.at[:, pl.ds(cb, W)], ublk0)

        @pl.when(r >= TAIL)
        def _():
            pltpu.sync_copy(tbl.at[:, pl.ds(TAIL, TW)], tailb)

        colm = jnp.zeros((16,), jnp.int32) + (r % W)
        colt = jnp.clip(jnp.zeros((16,), jnp.int32) + (r - TAIL), 0, TW - 1)
        pieces = []
        for j in range(8):
            m = plsc.load_gather(ublk0, [lane + j * 16, colm])
            tl = plsc.load_gather(tailb, [lane + j * 16, colt])
            pieces.append(jnp.where(r < TAIL, m, tl))
        m = plsc.load_gather(ublk0, [d128, colm])
        tl = plsc.load_gather(tailb, [d128, colt])
        pieces.append(jnp.where(r < TAIL, m, tl))
        return pieces

    def ep_step(b, carry):
        ok, ru, ri = pair_normal(b)

        @pl.when(jnp.logical_not(ok))
        def _():
            up = fetch_one(ru, uT_hbm)
            ip = fetch_one(ri, iT_hbm)
            s = dot(up, ip)
            gb = (b // 16) * 16
            old = outc[pl.ds(gb, 16)]
            outc[pl.ds(gb, 16)] = jnp.where(lane == (b % 16), s, old)

        return carry

    lax.fori_loop(0, BPW, ep_step, 0)
    pltpu.sync_copy(outc, out_hbm.at[pl.ds(base, BPW)])


def kernel(users, items, user_emb, item_emb):
    perm = jnp.argsort(users)
    tmp = _sc_dot2(users[perm], items[perm], user_emb.T, item_emb.T)
    return jnp.zeros((B,), jnp.float32).at[perm].set(tmp)
